# baseline (speedup 1.0000x reference)
import functools

import jax
import jax.numpy as jnp
from jax import lax
from jax.experimental import pallas as pl
from jax.experimental.pallas import tpu as pltpu
from jax.experimental.pallas import tpu_sc as plsc

B = 10
N_PER = 10000
K = 100
D = 128

N_PAD = 10240
_SC_CORES = 2
_SC_SUBCORES = 16
_NW = _SC_CORES * _SC_SUBCORES
_GPC = B // _SC_CORES
_CPW = N_PAD // _SC_SUBCORES
_NV = _CPW // 16
_SEL_SLOTS = 128
_SEL_TOTAL = B * _SEL_SLOTS + _NW
_GATHER_ROWS = 1024
_ROWS_PER_W = _GATHER_ROWS // _NW


def _rowmax_body(x_ref, o_ref):
    m = jnp.max(x_ref[...], axis=1)
    bits = lax.bitcast_convert_type(m, jnp.uint32)
    neg = (bits >> jnp.uint32(31)) != 0
    mono = jnp.where(neg, ~bits, bits | jnp.uint32(0x80000000))
    mono = jnp.concatenate([mono, jnp.zeros((N_PAD - N_PER,), jnp.uint32)])
    o_ref[...] = mono.reshape(1, 1, N_PAD)


def _rowmax(feat):
    return pl.pallas_call(
        _rowmax_body,
        grid=(B,),
        in_specs=[pl.BlockSpec((N_PER, D), lambda g: (g, 0))],
        out_specs=pl.BlockSpec((1, 1, N_PAD), lambda g: (g, 0, 0)),
        out_shape=jax.ShapeDtypeStruct((B, 1, N_PAD), jnp.uint32),
    )(feat)


def _lane_iota():
    return lax.iota(jnp.int32, 16)


def _splat(x):
    return jnp.broadcast_to(x, (16,))


def _select_body(ku_hbm, selk_hbm, seli_hbm,
                 keys_v, hist_v, merged_v, allh_v,
                 counts_v, counts_all_v, listk_v, listi_v, pos_v,
                 rem_s, pkey_s, ipfx_s,
                 hist_sh, counts_sh, sem):
    c = lax.axis_index("c")
    s = lax.axis_index("s")
    wid = s * _SC_CORES + c
    lane = _lane_iota()

    zeros16 = jnp.zeros((16,), jnp.int32)
    ones16 = lane == lane

    for gl in range(_GPC):
        g = _GPC * c + gl
        pltpu.sync_copy(ku_hbm.at[pl.ds(g * N_PAD + _CPW * s, _CPW)],
                        keys_v.at[pl.ds(gl * _CPW, _CPW)])

    for gl in range(_GPC):
        rem_s[gl] = jnp.int32(K)
        pkey_s[gl] = jnp.int32(0)
        ipfx_s[gl] = jnp.int32(0)

    n_bins = _GPC * 256

    def hzero(i, _):
        hist_v[pl.ds(i * 16, 16)] = zeros16
        return 0

    for p in range(6):
        lax.fori_loop(0, n_bins // 16, hzero, 0)

        for gl in range(_GPC):
            if p == 0:
                pfx_u = None
            elif p <= 3:
                pfx_u = _splat(pkey_s[gl]).astype(jnp.uint32)
            else:
                tkey_u = plsc.bitcast(_splat(pkey_s[gl]), jnp.uint32)
                ipfx_v = _splat(ipfx_s[gl])

            def hist_body(v, _):
                k = keys_v[pl.ds(gl * _CPW + v * 16, 16)]
                col = _CPW * s + v * 16 + lane
                idxc = jnp.int32(16383) - col
                if p == 0:
                    d = (k >> jnp.uint32(24)).astype(jnp.int32)
                    act = col == col
                elif p <= 3:
                    sh = 24 - 8 * p
                    act = (k >> jnp.uint32(sh + 8)) == pfx_u
                    d = ((k >> jnp.uint32(sh)) & jnp.uint32(0xFF)).astype(jnp.int32)
                elif p == 4:
                    d = idxc >> 7
                    act = k == tkey_u
                else:
                    d = idxc & jnp.int32(0x7F)
                    act = (k == tkey_u) & ((idxc >> 7) == ipfx_v)
                plsc.addupdate_scatter(hist_v, [gl * 256 + d],
                                       jnp.ones((16,), jnp.int32), mask=act)
                return 0
            lax.fori_loop(0, _NV, hist_body, 0)

        buf = (p % 2) * _SC_SUBCORES * n_bins
        pltpu.sync_copy(hist_v,
                        hist_sh.at[pl.ds(buf + s * n_bins, n_bins)])
        plsc.subcore_barrier()
        pltpu.sync_copy(hist_sh.at[pl.ds(buf, _SC_SUBCORES * n_bins)],
                        allh_v)

        def msum(i, _):
            acc = allh_v[pl.ds(i * 16, 16)]
            for w in range(1, _SC_SUBCORES):
                acc = acc + allh_v[pl.ds(w * n_bins + i * 16, 16)]
            merged_v[pl.ds(i * 16, 16)] = acc
            return 0
        lax.fori_loop(0, n_bins // 16, msum, 0)

        for gl in range(_GPC):
            rem0 = rem_s[gl]
            remv = _splat(rem0)
            rowtot = jnp.zeros((16,), jnp.int32)
            for r in range(16):
                tr = jnp.sum(merged_v[pl.ds(gl * 256 + r * 16, 16)])
                rowtot = jnp.where(lane == r, _splat(tr), rowtot)
            row_sfx = lax.rev(plsc.cumsum(lax.rev(rowtot, (0,))), (0,))
            trow = plsc.all_reduce_population_count(row_sfx >= remv)[0] - 1
            pickr = lane == trow
            carry = (jnp.sum(jnp.where(pickr, row_sfx, 0))
                     - jnp.sum(jnp.where(pickr, rowtot, 0)))
            vrow = merged_v[pl.ds(gl * 256 + trow * 16, 16)]
            in_sfx = (lax.rev(plsc.cumsum(lax.rev(vrow, (0,))), (0,))
                      + _splat(carry))
            tlane = plsc.all_reduce_population_count(in_sfx >= remv)[0] - 1
            t = trow * 16 + tlane
            pick = lane == tlane
            si_t = jnp.sum(jnp.where(pick, in_sfx, 0))
            h_t = jnp.sum(jnp.where(pick, vrow, 0))
            rem_s[gl] = rem0 - (si_t - h_t)
            if p <= 3:
                pkey_sh = pkey_s[gl]
                pkey_s[gl] = pkey_sh * 256 + t
            elif p == 4:
                ipfx_s[gl] = t
            else:
                ipfx_s[gl] = ipfx_s[gl] * 128 + t
        plsc.subcore_barrier()

    counts = jnp.zeros((16,), jnp.int32)
    for gl in range(_GPC):
        tkey = plsc.bitcast(_splat(pkey_s[gl]), jnp.uint32)
        tixc = _splat(ipfx_s[gl])

        def cnt_body(v, acc):
            k = keys_v[pl.ds(gl * _CPW + v * 16, 16)]
            col = _CPW * s + v * 16 + lane
            idxc = jnp.int32(16383) - col
            sel = (k > tkey) | ((k == tkey) & (idxc >= tixc))
            return acc + plsc.all_reduce_population_count(sel)

        cg = lax.fori_loop(0, _NV, cnt_body, jnp.zeros((16,), jnp.int32))
        counts = jnp.where(lane == gl, cg, counts)

    counts_v[...] = counts
    pltpu.sync_copy(counts_v, counts_sh.at[pl.ds(s * 16, 16)])
    plsc.subcore_barrier()
    pltpu.sync_copy(counts_sh, counts_all_v)

    base = jnp.zeros((16,), jnp.int32)
    for w in range(_SC_SUBCORES):
        base = base + jnp.where(jnp.int32(w) < s,
                                counts_all_v[pl.ds(w * 16, 16)],
                                jnp.zeros((16,), jnp.int32))

    def pinit(i, _):
        pos_v[pl.ds(i * 16, 16)] = _splat(B * _SEL_SLOTS + wid)
        listk_v[pl.ds(i * 16, 16)] = jnp.zeros((16,), jnp.int32)
        listi_v[pl.ds(i * 16, 16)] = jnp.zeros((16,), jnp.int32)
        return 0
    lax.fori_loop(0, 8, pinit, 0)

    lcount = jnp.zeros((16,), jnp.int32)
    for gl in range(_GPC):
        g = _GPC * c + gl
        tkey = plsc.bitcast(_splat(pkey_s[gl]), jnp.uint32)
        tixc = _splat(ipfx_s[gl])
        cbase = _splat(base[gl])

        def col_body(v, carry):
            lcnt, cgraph = carry
            k = keys_v[pl.ds(gl * _CPW + v * 16, 16)]
            col = _CPW * s + v * 16 + lane
            idxc = jnp.int32(16383) - col
            sel = (k > tkey) | ((k == tkey) & (idxc >= tixc))
            csum = plsc.cumsum(sel.astype(jnp.int32))
            lslot = jnp.minimum(lcnt + csum - 1, jnp.int32(127))
            ksort = plsc.bitcast(k ^ jnp.uint32(0x80000000), jnp.int32)
            gid = jnp.int32(g * N_PER) + col
            hpos = jnp.int32(g * _SEL_SLOTS) + cbase + cgraph + csum - 1
            plsc.store_scatter(listk_v, [lslot], ksort, mask=sel)
            plsc.store_scatter(listi_v, [lslot], gid, mask=sel)
            plsc.store_scatter(pos_v, [lslot], hpos, mask=sel)
            pop = plsc.all_reduce_population_count(sel)
            return lcnt + pop, cgraph + pop

        lcount, _ = lax.fori_loop(0, _NV, col_body,
                                  (lcount, jnp.zeros((16,), jnp.int32)))

    pltpu.async_copy(listk_v, selk_hbm.at[pos_v], sem).wait()
    pltpu.async_copy(listi_v, seli_hbm.at[pos_v], sem).wait()


def _sc_select(ku_flat):
    mesh = plsc.VectorSubcoreMesh(core_axis_name="c", subcore_axis_name="s")
    kern = functools.partial(
        pl.kernel,
        mesh=mesh,
        compiler_params=pltpu.CompilerParams(needs_layout_passes=False),
        out_type=(jax.ShapeDtypeStruct((_SEL_TOTAL,), jnp.int32),
                  jax.ShapeDtypeStruct((_SEL_TOTAL,), jnp.int32)),
        scratch_types=[
            pltpu.VMEM((_GPC * _CPW,), jnp.uint32),
            pltpu.VMEM((_GPC * 256,), jnp.int32),
            pltpu.VMEM((_GPC * 256,), jnp.int32),
            pltpu.VMEM((_SC_SUBCORES * _GPC * 256,), jnp.int32),
            pltpu.VMEM((16,), jnp.int32),
            pltpu.VMEM((_SC_SUBCORES * 16,), jnp.int32),
            pltpu.VMEM((_SEL_SLOTS,), jnp.int32),
            pltpu.VMEM((_SEL_SLOTS,), jnp.int32),
            pltpu.VMEM((_SEL_SLOTS,), jnp.int32),
            pltpu.SMEM((16,), jnp.int32),
            pltpu.SMEM((16,), jnp.int32),
            pltpu.SMEM((16,), jnp.int32),
            pltpu.VMEM_SHARED((2 * _SC_SUBCORES * _GPC * 256,), jnp.int32),
            pltpu.VMEM_SHARED((_SC_SUBCORES * 16,), jnp.int32),
            pltpu.SemaphoreType.DMA,
        ],
    )(_select_body)
    return kern(ku_flat)


def _order_body(k_ref, i_ref, o_ref):
    kk = k_ref[...]
    ii = i_ref[...]
    lane = lax.broadcasted_iota(jnp.int32, (B, _SEL_SLOTS), 1)
    kk = jnp.where(lane >= K, jnp.int32(-2147483647 - 1), kk)
    ii = jnp.where(lane >= K, jnp.int32(1 << 29) + lane, ii)
    n = _SEL_SLOTS
    k = 2
    while k <= n:
        j = k // 2
        while j >= 1:
            bitj = (lane & j) != 0
            pk = jnp.where(bitj, jnp.roll(kk, j, axis=1), jnp.roll(kk, -j, axis=1))
            pi = jnp.where(bitj, jnp.roll(ii, j, axis=1), jnp.roll(ii, -j, axis=1))
            want_first = ((lane & k) == 0) == ~bitj
            first = (kk > pk) | ((kk == pk) & (ii < pi))
            keep = first == want_first
            kk = jnp.where(keep, kk, pk)
            ii = jnp.where(keep, ii, pi)
            j //= 2
        k *= 2
    o_ref[...] = ii


def _order(selk2d, seli2d):
    return pl.pallas_call(
        _order_body,
        out_shape=jax.ShapeDtypeStruct((B, _SEL_SLOTS), jnp.int32),
    )(selk2d, seli2d)


def _gather_body(feat_hbm, idx_hbm, out_hbm, idx_v, rows_v, sem):
    wid = lax.axis_index("s") * _SC_CORES + lax.axis_index("c")
    base = wid * _ROWS_PER_W
    pltpu.sync_copy(idx_hbm.at[pl.ds(base, _ROWS_PER_W)], idx_v)
    pltpu.async_copy(feat_hbm.at[idx_v], rows_v, sem).wait()
    pltpu.sync_copy(rows_v, out_hbm.at[pl.ds(base, _ROWS_PER_W)])


def _gather_rows(feat, idx_flat):
    mesh = plsc.VectorSubcoreMesh(core_axis_name="c", subcore_axis_name="s")
    kern = functools.partial(
        pl.kernel,
        mesh=mesh,
        out_type=jax.ShapeDtypeStruct((_GATHER_ROWS, D), jnp.float32),
        scratch_types=[
            pltpu.VMEM((_ROWS_PER_W,), jnp.int32),
            pltpu.VMEM((_ROWS_PER_W, D), jnp.float32),
            pltpu.SemaphoreType.DMA,
        ],
    )(_gather_body)
    return kern(feat, idx_flat)


def _rowsort_body(x_ref, o_ref):
    x = x_ref[...]
    i = lax.broadcasted_iota(jnp.int32, x.shape, 1)
    k = 2
    while k <= D:
        j = k // 2
        while j >= 1:
            bitj = (i & j) != 0
            p = jnp.where(bitj, jnp.roll(x, j, axis=1), jnp.roll(x, -j, axis=1))
            want_first = ((i & k) == 0) == ~bitj
            keep = (x <= p) == want_first
            x = jnp.where(keep, x, p)
            j //= 2
        k *= 2
    o_ref[...] = x


def _rowsort(rows):
    return pl.pallas_call(
        _rowsort_body,
        out_shape=jax.ShapeDtypeStruct((_GATHER_ROWS, D), jnp.float32),
    )(rows)


def kernel(feat):
    ku = _rowmax(feat).reshape(B * N_PAD)
    selk, seli = _sc_select(ku)
    selk2d = selk[: B * _SEL_SLOTS].reshape(B, _SEL_SLOTS)
    seli2d = seli[: B * _SEL_SLOTS].reshape(B, _SEL_SLOTS)
    ranked = _order(selk2d, seli2d)
    idx_flat = ranked[:, :K].reshape(B * K)
    pad = jnp.arange(_GATHER_ROWS - B * K, dtype=jnp.int32)
    idx_flat = jnp.concatenate([idx_flat, pad])
    rows = _gather_rows(feat, idx_flat)
    srt = _rowsort(rows)
    return srt[: B * K].reshape(B, K * D)


# speedup vs baseline: 4.6718x; 4.6718x over previous
import functools

import jax
import jax.numpy as jnp
from jax import lax
from jax.experimental import pallas as pl
from jax.experimental.pallas import tpu as pltpu
from jax.experimental.pallas import tpu_sc as plsc

B = 10
N_PER = 10000
K = 100
D = 128

_SC_CORES = 2
_SC_SUBCORES = 16
_NW = _SC_CORES * _SC_SUBCORES
_GATHER_ROWS = 1024
_ROWS_PER_W = _GATHER_ROWS // _NW


def _rowmax_body(x_ref, o_ref):
    m = jnp.max(x_ref[...], axis=1)
    o_ref[...] = m.reshape(1, 1, N_PER)


def _rowmax(feat):
    return pl.pallas_call(
        _rowmax_body,
        grid=(B,),
        in_specs=[pl.BlockSpec((N_PER, D), lambda g: (g, 0))],
        out_specs=pl.BlockSpec((1, 1, N_PER), lambda g: (g, 0, 0)),
        out_shape=jax.ShapeDtypeStruct((B, 1, N_PER), jnp.float32),
    )(feat)


def _topk_body(keys_ref, idx_ref):
    keys = keys_ref[...]
    lane = lax.broadcasted_iota(jnp.int32, (B, N_PER), 1)
    lane128 = lax.broadcasted_iota(jnp.int32, (B, 128), 1)

    def body(t, carry):
        kc, acc = carry
        m = jnp.max(kc, axis=1, keepdims=True)
        idx = jnp.min(
            jnp.where(kc == m, lane, jnp.int32(1 << 30)), axis=1, keepdims=True
        )
        acc = jnp.where(lane128 == t, idx, acc)
        kc = jnp.where(lane == idx, jnp.float32(-jnp.inf), kc)
        return kc, acc

    _, acc = lax.fori_loop(0, K, body, (keys, lane128))
    row = lax.broadcasted_iota(jnp.int32, (B, 128), 0)
    idx_ref[...] = acc + N_PER * row


def _topk(keys2d):
    return pl.pallas_call(
        _topk_body,
        out_shape=jax.ShapeDtypeStruct((B, 128), jnp.int32),
    )(keys2d)


def _gather_body(feat_hbm, idx_hbm, out_hbm, idx_v, rows_v, sem):
    wid = lax.axis_index("s") * _SC_CORES + lax.axis_index("c")
    base = wid * _ROWS_PER_W
    pltpu.sync_copy(idx_hbm.at[pl.ds(base, _ROWS_PER_W)], idx_v)
    pltpu.async_copy(feat_hbm.at[idx_v], rows_v, sem).wait()
    pltpu.sync_copy(rows_v, out_hbm.at[pl.ds(base, _ROWS_PER_W)])


def _gather_rows(feat, idx_flat):
    mesh = plsc.VectorSubcoreMesh(core_axis_name="c", subcore_axis_name="s")
    kern = functools.partial(
        pl.kernel,
        mesh=mesh,
        out_type=jax.ShapeDtypeStruct((_GATHER_ROWS, D), jnp.float32),
        scratch_types=[
            pltpu.VMEM((_ROWS_PER_W,), jnp.int32),
            pltpu.VMEM((_ROWS_PER_W, D), jnp.float32),
            pltpu.SemaphoreType.DMA,
        ],
    )(_gather_body)
    return kern(feat, idx_flat)


def _rowsort_body(x_ref, o_ref):
    x = x_ref[...]
    i = lax.broadcasted_iota(jnp.int32, x.shape, 1)
    k = 2
    while k <= D:
        j = k // 2
        while j >= 1:
            bitj = (i & j) != 0
            p = jnp.where(bitj, jnp.roll(x, j, axis=1), jnp.roll(x, -j, axis=1))
            want_first = ((i & k) == 0) == ~bitj
            keep = (x <= p) == want_first
            x = jnp.where(keep, x, p)
            j //= 2
        k *= 2
    o_ref[...] = x


def _rowsort(rows):
    return pl.pallas_call(
        _rowsort_body,
        out_shape=jax.ShapeDtypeStruct((_GATHER_ROWS, D), jnp.float32),
    )(rows)


def kernel(feat):
    keys2d = _rowmax(feat).reshape(B, N_PER)
    idx = _topk(keys2d)
    idx_flat = idx[:, :K].reshape(B * K)
    pad = jnp.arange(_GATHER_ROWS - B * K, dtype=jnp.int32)
    idx_flat = jnp.concatenate([idx_flat, pad])
    rows = _gather_rows(feat, idx_flat)
    srt = _rowsort(rows)
    return srt[: B * K].reshape(B, K * D)
